# baseline (speedup 1.0000x reference)
import functools

import jax
import jax.numpy as jnp
from jax import lax
from jax.experimental import pallas as pl
from jax.experimental.pallas import tpu as pltpu
from jax.experimental.pallas import tpu_sc as plsc

HIDDEN = 512
TDIM = 64
REL_DIM = 16
N_NBR = 8192
LANES = 16
NC = 2
NS = 16
NW = NC * NS
PER_W = N_NBR // NW
CHUNK_SIZES = (96, 96, 64)
CHUNK_OFFS = (0, 96, 192)
CHUNK = max(CHUNK_SIZES)
NCHUNK = len(CHUNK_SIZES)
VECS = HIDDEN // LANES


def _sc_body(v_hbm, nbr_hbm, t_hbm, se_hbm, out_hbm,
             idx0, idx1, idx2, t_v, rows0, rows1, acc, se_v, v0,
             sem0, sem1, sem2):
    cid = lax.axis_index("c")
    sid = lax.axis_index("s")
    wid = sid * NC + cid
    base = wid * PER_W

    idx_refs = [idx0, idx1, idx2]
    cps = [
        pltpu.make_async_copy(
            nbr_hbm.at[pl.ds(base + CHUNK_OFFS[c], CHUNK_SIZES[c])],
            idx_refs[c], sem2)
        for c in range(NCHUNK)
    ] + [
        pltpu.make_async_copy(t_hbm.at[pl.ds(base, PER_W)], t_v, sem2),
        pltpu.make_async_copy(se_hbm, se_v, sem2),
        pltpu.make_async_copy(v_hbm.at[pl.ds(0, 1)], v0, sem2),
    ]
    for cp in cps:
        cp.start()
    for cp in cps:
        cp.wait()

    s_vec = se_v[0, :]
    e_vec = se_v[1, :]
    zero_i = jnp.zeros((LANES,), jnp.int32)
    n_out_vec = jnp.zeros((LANES,), jnp.int32)
    zero = jnp.zeros((LANES,), jnp.float32)
    for c in range(NCHUNK):
        for i in range(CHUNK_SIZES[c] // LANES):
            gi = CHUNK_OFFS[c] + i * LANES
            t = t_v[pl.ds(gi, LANES)]
            m = jnp.logical_and(t >= s_vec, t < e_vec)
            nb = idx_refs[c][pl.ds(i * LANES, LANES)]
            idx_refs[c][pl.ds(i * LANES, LANES)] = jnp.where(m, nb, zero_i)
            n_out_vec = n_out_vec + plsc.all_reduce_population_count(
                jnp.logical_not(m))
    n_out = n_out_vec.astype(jnp.float32)

    rows = [rows0, rows1]
    sems = [sem0, sem1]
    copies = [
        pltpu.make_async_copy(
            v_hbm.at[idx_refs[c]],
            rows[c % 2].at[pl.ds(0, CHUNK_SIZES[c])], sems[c % 2])
        for c in range(NCHUNK)
    ]
    copies[0].start()
    ngrp, grp, runroll = 4, VECS // 4, 4
    accs = [tuple(zero for _ in range(grp)) for _ in range(ngrp)]
    for c in range(NCHUNK):
        if c + 1 < NCHUNK:
            copies[c + 1].start()
        copies[c].wait()
        buf = rows[c % 2]

        for h in range(ngrp):
            def body(ri, carry, buf=buf, h=h):
                r = ri * runroll
                for u in range(runroll):
                    carry = tuple(
                        a + buf[r + u, pl.ds((h * grp + k) * LANES, LANES)]
                        for k, a in enumerate(carry))
                return carry

            accs[h] = lax.fori_loop(0, CHUNK_SIZES[c] // runroll, body, accs[h])

    for k in range(VECS):
        acc[0, pl.ds(k * LANES, LANES)] = (
            accs[k // grp][k % grp]
            - n_out * v0[0, pl.ds(k * LANES, LANES)])

    pltpu.sync_copy(acc, out_hbm.at[pl.ds(wid, 1)])


def _sc_gather_sum(v_, nbr, t_flat, se16):
    mesh = plsc.VectorSubcoreMesh(
        core_axis_name="c", subcore_axis_name="s", num_cores=NC,
        num_subcores=NS)
    fn = pl.kernel(
        _sc_body,
        out_type=jax.ShapeDtypeStruct((NW, HIDDEN), jnp.float32),
        mesh=mesh,
        compiler_params=pltpu.CompilerParams(needs_layout_passes=False),
        scratch_types=[
            pltpu.VMEM((CHUNK_SIZES[0],), jnp.int32),
            pltpu.VMEM((CHUNK_SIZES[1],), jnp.int32),
            pltpu.VMEM((CHUNK_SIZES[2],), jnp.int32),
            pltpu.VMEM((PER_W,), jnp.float32),
            pltpu.VMEM((CHUNK, HIDDEN), jnp.float32),
            pltpu.VMEM((CHUNK, HIDDEN), jnp.float32),
            pltpu.VMEM((1, HIDDEN), jnp.float32),
            pltpu.VMEM((2, LANES), jnp.float32),
            pltpu.VMEM((1, HIDDEN), jnp.float32),
            pltpu.SemaphoreType.DMA,
            pltpu.SemaphoreType.DMA,
            pltpu.SemaphoreType.DMA,
        ],
    )
    return fn(v_, nbr, t_flat, se16)


def _tc_body(trow_ref, t64_ref, rels_ref, wb_ref, twv_ref, ewv_ref,
             tbv_ref, ebv_ref, params_ref, out_ref):
    s = params_ref[0, 0]
    e = params_ref[0, 1]
    w0 = params_ref[0, 2]
    b0 = params_ref[0, 3]
    trow = trow_ref[...]
    mrow = jnp.logical_and(trow >= s, trow < e).astype(jnp.float32)
    cnt = jnp.sum(mrow)
    s_time = jnp.sum(trow * mrow)
    t64 = t64_ref[...]
    m64 = jnp.logical_and(t64 >= s, t64 < e).astype(jnp.float32)
    rowids = lax.broadcasted_iota(jnp.int32, (TDIM, 128), 0)
    facc = jnp.zeros((TDIM, 128), jnp.float32)
    for j in range(1, TDIM):
        pj = m64 * jnp.sin(t64 * wb_ref[0, j] + wb_ref[1, j])
        pr = jnp.sum(pj, axis=0, keepdims=True)
        facc = jnp.where(rowids == j, pr, facc)
    lanesum = jnp.sum(facc, axis=1, keepdims=True)
    col_ids = lax.broadcasted_iota(jnp.int32, (TDIM, 1), 0)
    s_feat_col = jnp.where(col_ids == 0, w0 * s_time + b0 * cnt, lanesum)
    s_rel = lax.dot_general(
        mrow, rels_ref[...], (((1,), (1,)), ((), ())),
        preferred_element_type=jnp.float32)
    proj_t = lax.dot_general(
        s_feat_col, twv_ref[...], (((0,), (0,)), ((), ())),
        preferred_element_type=jnp.float32)
    out_ref[...] = (
        proj_t
        + jnp.dot(s_rel, ewv_ref[...], preferred_element_type=jnp.float32)
        + cnt * (tbv_ref[...] + ebv_ref[...]))


def kernel(nid, k_, q_, v_, start_t, end_t, neighbors, times, rels,
           t2v_w0, t2v_b0, t2v_W, t2v_B,
           tWk, tbk, tWq, tbq, tWv, tbv,
           eWk, ebk, eWq, ebq, eWv, ebv):
    del nid, k_, q_, tWk, tbk, tWq, tbq, eWk, ebk, eWq, ebq
    s_f = jnp.asarray(start_t, jnp.float32)
    e_f = jnp.asarray(end_t, jnp.float32)
    t_flat = jnp.reshape(times, (N_NBR,)).astype(jnp.float32)
    nbr = jnp.reshape(neighbors, (N_NBR,)).astype(jnp.int32)
    se16 = jnp.broadcast_to(
        jnp.stack([s_f, e_f])[:, None], (2, LANES))

    scpart = _sc_gather_sum(v_, nbr, t_flat, se16)

    wfull = jnp.concatenate(
        [jnp.zeros((1, 1), jnp.float32), t2v_W], axis=1)
    bfull = jnp.concatenate(
        [jnp.zeros((1,), jnp.float32), t2v_B], axis=0)[None, :]
    wb = jnp.concatenate([wfull, bfull], axis=0)
    params = jnp.stack(
        [s_f, e_f, t2v_w0[0, 0], t2v_b0[0]]).reshape(1, 4)

    tc_out = pl.pallas_call(
        _tc_body,
        out_shape=jax.ShapeDtypeStruct((1, HIDDEN), jnp.float32),
    )(jnp.reshape(times, (1, N_NBR)), jnp.reshape(times, (64, 128)),
      jnp.transpose(rels), wb, tWv, eWv, tbv[None, :], ebv[None, :], params)
    return jnp.sum(jnp.concatenate([scpart, tc_out], axis=0),
                   axis=0, keepdims=True)


# speedup vs baseline: 1.0089x; 1.0089x over previous
import functools

import jax
import jax.numpy as jnp
from jax import lax
from jax.experimental import pallas as pl
from jax.experimental.pallas import tpu as pltpu
from jax.experimental.pallas import tpu_sc as plsc

HIDDEN = 512
TDIM = 64
REL_DIM = 16
N_NBR = 8192
LANES = 16
NC = 2
NS = 16
NW = NC * NS
PER_W = N_NBR // NW
CHUNK_SIZES = (96, 96, 64)
CHUNK_OFFS = (0, 96, 192)
CHUNK = max(CHUNK_SIZES)
NCHUNK = len(CHUNK_SIZES)
VECS = HIDDEN // LANES


def _sc_body(v_hbm, nbr_hbm, t_hbm, se_hbm, out_hbm,
             idx0, idx1, idx2, t_v, rows0, rows1, acc, se_v, v0,
             sem0, sem1, sem2):
    cid = lax.axis_index("c")
    sid = lax.axis_index("s")
    wid = sid * NC + cid
    base = wid * PER_W

    idx_refs = [idx0, idx1, idx2]
    cps = [
        pltpu.make_async_copy(
            nbr_hbm.at[pl.ds(base + CHUNK_OFFS[c], CHUNK_SIZES[c])],
            idx_refs[c], sem2)
        for c in range(NCHUNK)
    ] + [
        pltpu.make_async_copy(t_hbm.at[pl.ds(base, PER_W)], t_v, sem2),
        pltpu.make_async_copy(se_hbm, se_v, sem2),
        pltpu.make_async_copy(v_hbm.at[pl.ds(0, 1)], v0, sem2),
    ]
    for cp in cps:
        cp.start()
    for cp in cps[:-1]:
        cp.wait()

    s_vec = se_v[0, :]
    e_vec = se_v[1, :]
    zero_i = jnp.zeros((LANES,), jnp.int32)
    zero = jnp.zeros((LANES,), jnp.float32)
    n_out_vecs = [jnp.zeros((LANES,), jnp.int32)]

    def adjust(c):
        for i in range(CHUNK_SIZES[c] // LANES):
            gi = CHUNK_OFFS[c] + i * LANES
            t = t_v[pl.ds(gi, LANES)]
            m = jnp.logical_and(t >= s_vec, t < e_vec)
            nb = idx_refs[c][pl.ds(i * LANES, LANES)]
            idx_refs[c][pl.ds(i * LANES, LANES)] = jnp.where(m, nb, zero_i)
            n_out_vecs[0] = n_out_vecs[0] + plsc.all_reduce_population_count(
                jnp.logical_not(m))

    rows = [rows0, rows1]
    sems = [sem0, sem1]
    copies = [
        pltpu.make_async_copy(
            v_hbm.at[idx_refs[c]],
            rows[c % 2].at[pl.ds(0, CHUNK_SIZES[c])], sems[c % 2])
        for c in range(NCHUNK)
    ]
    adjust(0)
    copies[0].start()
    ngrp, grp, runroll = 4, VECS // 4, 4
    accs = [tuple(zero for _ in range(grp)) for _ in range(ngrp)]
    for c in range(NCHUNK):
        if c + 1 < NCHUNK:
            adjust(c + 1)
            copies[c + 1].start()
        copies[c].wait()
        buf = rows[c % 2]

        for h in range(ngrp):
            def body(ri, carry, buf=buf, h=h):
                r = ri * runroll
                for u in range(runroll):
                    carry = tuple(
                        a + buf[r + u, pl.ds((h * grp + k) * LANES, LANES)]
                        for k, a in enumerate(carry))
                return carry

            accs[h] = lax.fori_loop(0, CHUNK_SIZES[c] // runroll, body, accs[h])

    cps[-1].wait()
    n_out = n_out_vecs[0].astype(jnp.float32)
    for k in range(VECS):
        acc[0, pl.ds(k * LANES, LANES)] = (
            accs[k // grp][k % grp]
            - n_out * v0[0, pl.ds(k * LANES, LANES)])

    pltpu.sync_copy(acc, out_hbm.at[pl.ds(wid, 1)])


def _sc_gather_sum(v_, nbr, t_flat, se16):
    mesh = plsc.VectorSubcoreMesh(
        core_axis_name="c", subcore_axis_name="s", num_cores=NC,
        num_subcores=NS)
    fn = pl.kernel(
        _sc_body,
        out_type=jax.ShapeDtypeStruct((NW, HIDDEN), jnp.float32),
        mesh=mesh,
        compiler_params=pltpu.CompilerParams(needs_layout_passes=False),
        scratch_types=[
            pltpu.VMEM((CHUNK_SIZES[0],), jnp.int32),
            pltpu.VMEM((CHUNK_SIZES[1],), jnp.int32),
            pltpu.VMEM((CHUNK_SIZES[2],), jnp.int32),
            pltpu.VMEM((PER_W,), jnp.float32),
            pltpu.VMEM((CHUNK, HIDDEN), jnp.float32),
            pltpu.VMEM((CHUNK, HIDDEN), jnp.float32),
            pltpu.VMEM((1, HIDDEN), jnp.float32),
            pltpu.VMEM((2, LANES), jnp.float32),
            pltpu.VMEM((1, HIDDEN), jnp.float32),
            pltpu.SemaphoreType.DMA,
            pltpu.SemaphoreType.DMA,
            pltpu.SemaphoreType.DMA,
        ],
    )
    return fn(v_, nbr, t_flat, se16)


def _tc_body(trow_ref, t64_ref, rels_ref, wb_ref, twv_ref, ewv_ref,
             tbv_ref, ebv_ref, params_ref, out_ref):
    s = params_ref[0, 0]
    e = params_ref[0, 1]
    w0 = params_ref[0, 2]
    b0 = params_ref[0, 3]
    trow = trow_ref[...]
    mrow = jnp.logical_and(trow >= s, trow < e).astype(jnp.float32)
    cnt = jnp.sum(mrow)
    s_time = jnp.sum(trow * mrow)
    t64 = t64_ref[...]
    m64 = jnp.logical_and(t64 >= s, t64 < e).astype(jnp.float32)
    rowids = lax.broadcasted_iota(jnp.int32, (TDIM, 128), 0)
    facc = jnp.zeros((TDIM, 128), jnp.float32)
    for j in range(1, TDIM):
        pj = m64 * jnp.sin(t64 * wb_ref[0, j] + wb_ref[1, j])
        pr = jnp.sum(pj, axis=0, keepdims=True)
        facc = jnp.where(rowids == j, pr, facc)
    lanesum = jnp.sum(facc, axis=1, keepdims=True)
    col_ids = lax.broadcasted_iota(jnp.int32, (TDIM, 1), 0)
    s_feat_col = jnp.where(col_ids == 0, w0 * s_time + b0 * cnt, lanesum)
    s_rel = lax.dot_general(
        mrow, rels_ref[...], (((1,), (1,)), ((), ())),
        preferred_element_type=jnp.float32)
    proj_t = lax.dot_general(
        s_feat_col, twv_ref[...], (((0,), (0,)), ((), ())),
        preferred_element_type=jnp.float32)
    out_ref[...] = (
        proj_t
        + jnp.dot(s_rel, ewv_ref[...], preferred_element_type=jnp.float32)
        + cnt * (tbv_ref[...] + ebv_ref[...]))


def kernel(nid, k_, q_, v_, start_t, end_t, neighbors, times, rels,
           t2v_w0, t2v_b0, t2v_W, t2v_B,
           tWk, tbk, tWq, tbq, tWv, tbv,
           eWk, ebk, eWq, ebq, eWv, ebv):
    del nid, k_, q_, tWk, tbk, tWq, tbq, eWk, ebk, eWq, ebq
    s_f = jnp.asarray(start_t, jnp.float32)
    e_f = jnp.asarray(end_t, jnp.float32)
    t_flat = jnp.reshape(times, (N_NBR,)).astype(jnp.float32)
    nbr = jnp.reshape(neighbors, (N_NBR,)).astype(jnp.int32)
    se16 = jnp.broadcast_to(
        jnp.stack([s_f, e_f])[:, None], (2, LANES))

    scpart = _sc_gather_sum(v_, nbr, t_flat, se16)

    wfull = jnp.concatenate(
        [jnp.zeros((1, 1), jnp.float32), t2v_W], axis=1)
    bfull = jnp.concatenate(
        [jnp.zeros((1,), jnp.float32), t2v_B], axis=0)[None, :]
    wb = jnp.concatenate([wfull, bfull], axis=0)
    params = jnp.stack(
        [s_f, e_f, t2v_w0[0, 0], t2v_b0[0]]).reshape(1, 4)

    tc_out = pl.pallas_call(
        _tc_body,
        out_shape=jax.ShapeDtypeStruct((1, HIDDEN), jnp.float32),
    )(jnp.reshape(times, (1, N_NBR)), jnp.reshape(times, (64, 128)),
      jnp.transpose(rels), wb, tWv, eWv, tbv[None, :], ebv[None, :], params)
    return jnp.sum(jnp.concatenate([scpart, tc_out], axis=0),
                   axis=0, keepdims=True)


# speedup vs baseline: 1.0094x; 1.0005x over previous
import jax
import jax.numpy as jnp
from jax import lax
from jax.experimental import pallas as pl
from jax.experimental.pallas import tpu as pltpu
from jax.experimental.pallas import tpu_sc as plsc

HIDDEN = 512
TDIM = 64
REL_DIM = 16
N_NBR = 8192
LANES = 16
NC = 2
NS = 16
NW = NC * NS
PER_W = N_NBR // NW
CHUNK_SIZES = (96, 96, 64)
CHUNK_OFFS = (0, 96, 192)
CHUNK = max(CHUNK_SIZES)
NCHUNK = len(CHUNK_SIZES)
VECS = HIDDEN // LANES


def _sc_body(v_hbm, nbr_hbm, t_hbm, se_hbm, out_hbm,
             idx0, idx1, idx2, t_v, rows0, rows1, acc, se_v, v0,
             sem0, sem1, sem2):
    cid = lax.axis_index("c")
    sid = lax.axis_index("s")
    wid = sid * NC + cid
    base = wid * PER_W

    idx_refs = [idx0, idx1, idx2]
    cps = [
        pltpu.make_async_copy(
            nbr_hbm.at[pl.ds(base + CHUNK_OFFS[c], CHUNK_SIZES[c])],
            idx_refs[c], sem2)
        for c in range(NCHUNK)
    ] + [
        pltpu.make_async_copy(t_hbm.at[pl.ds(base, PER_W)], t_v, sem2),
        pltpu.make_async_copy(se_hbm, se_v, sem2),
        pltpu.make_async_copy(v_hbm.at[pl.ds(0, 1)], v0, sem2),
    ]
    for cp in cps:
        cp.start()
    for cp in cps[:-1]:
        cp.wait()

    s_vec = se_v[0, :]
    e_vec = se_v[1, :]
    zero_i = jnp.zeros((LANES,), jnp.int32)
    zero = jnp.zeros((LANES,), jnp.float32)
    n_out_vecs = [jnp.zeros((LANES,), jnp.int32)]

    def adjust(c):
        for i in range(CHUNK_SIZES[c] // LANES):
            gi = CHUNK_OFFS[c] + i * LANES
            t = t_v[pl.ds(gi, LANES)]
            m = jnp.logical_and(t >= s_vec, t < e_vec)
            nb = idx_refs[c][pl.ds(i * LANES, LANES)]
            idx_refs[c][pl.ds(i * LANES, LANES)] = jnp.where(m, nb, zero_i)
            n_out_vecs[0] = n_out_vecs[0] + plsc.all_reduce_population_count(
                jnp.logical_not(m))

    rows = [rows0, rows1]
    sems = [sem0, sem1]
    copies = [
        pltpu.make_async_copy(
            v_hbm.at[idx_refs[c]],
            rows[c % 2].at[pl.ds(0, CHUNK_SIZES[c])], sems[c % 2])
        for c in range(NCHUNK)
    ]
    adjust(0)
    copies[0].start()
    ngrp, grp, runroll = 4, VECS // 4, 4
    accs = [tuple(zero for _ in range(grp)) for _ in range(ngrp)]
    for c in range(NCHUNK):
        if c + 1 < NCHUNK:
            adjust(c + 1)
            copies[c + 1].start()
        copies[c].wait()
        buf = rows[c % 2]

        for h in range(ngrp):
            def body(ri, carry, buf=buf, h=h):
                r = ri * runroll
                for u in range(runroll):
                    carry = tuple(
                        a + buf[r + u, pl.ds((h * grp + k) * LANES, LANES)]
                        for k, a in enumerate(carry))
                return carry

            accs[h] = lax.fori_loop(0, CHUNK_SIZES[c] // runroll, body, accs[h])

    cps[-1].wait()
    n_out = n_out_vecs[0].astype(jnp.float32)
    for k in range(VECS):
        acc[0, pl.ds(k * LANES, LANES)] = (
            accs[k // grp][k % grp]
            - n_out * v0[0, pl.ds(k * LANES, LANES)])

    pltpu.sync_copy(acc, out_hbm.at[pl.ds(wid, 1)])


def _sc_gather_sum(v_, nbr, t_flat, se16):
    mesh = plsc.VectorSubcoreMesh(
        core_axis_name="c", subcore_axis_name="s", num_cores=NC,
        num_subcores=NS)
    fn = pl.kernel(
        _sc_body,
        out_type=jax.ShapeDtypeStruct((NW, HIDDEN), jnp.float32),
        mesh=mesh,
        compiler_params=pltpu.CompilerParams(needs_layout_passes=False),
        scratch_types=[
            pltpu.VMEM((CHUNK_SIZES[0],), jnp.int32),
            pltpu.VMEM((CHUNK_SIZES[1],), jnp.int32),
            pltpu.VMEM((CHUNK_SIZES[2],), jnp.int32),
            pltpu.VMEM((PER_W,), jnp.float32),
            pltpu.VMEM((CHUNK, HIDDEN), jnp.float32),
            pltpu.VMEM((CHUNK, HIDDEN), jnp.float32),
            pltpu.VMEM((1, HIDDEN), jnp.float32),
            pltpu.VMEM((2, LANES), jnp.float32),
            pltpu.VMEM((1, HIDDEN), jnp.float32),
            pltpu.SemaphoreType.DMA,
            pltpu.SemaphoreType.DMA,
            pltpu.SemaphoreType.DMA,
        ],
    )
    return fn(v_, nbr, t_flat, se16)


def _tc_body(trow_ref, t64_ref, rels_ref, wb_ref, twv_ref, ewv_ref,
             tbv_ref, ebv_ref, params_ref, out_ref):
    s = params_ref[0, 0]
    e = params_ref[0, 1]
    w0 = params_ref[0, 2]
    b0 = params_ref[0, 3]
    trow = trow_ref[...]
    mrow = jnp.logical_and(trow >= s, trow < e).astype(jnp.float32)
    cnt = jnp.sum(mrow)
    s_time = jnp.sum(trow * mrow)
    t64 = t64_ref[...]
    m64 = jnp.logical_and(t64 >= s, t64 < e).astype(jnp.float32)
    rowids = lax.broadcasted_iota(jnp.int32, (TDIM, 128), 0)
    facc = jnp.zeros((TDIM, 128), jnp.float32)
    for j in range(1, TDIM):
        pj = m64 * jnp.sin(t64 * wb_ref[0, j] + wb_ref[1, j])
        pr = jnp.sum(pj, axis=0, keepdims=True)
        facc = jnp.where(rowids == j, pr, facc)
    lanesum = jnp.sum(facc, axis=1, keepdims=True)
    col_ids = lax.broadcasted_iota(jnp.int32, (TDIM, 1), 0)
    s_feat_col = jnp.where(col_ids == 0, w0 * s_time + b0 * cnt, lanesum)
    s_rel = lax.dot_general(
        mrow, rels_ref[...], (((1,), (1,)), ((), ())),
        preferred_element_type=jnp.float32)
    proj_t = lax.dot_general(
        s_feat_col, twv_ref[...], (((0,), (0,)), ((), ())),
        preferred_element_type=jnp.float32)
    out_ref[...] = (
        proj_t
        + jnp.dot(s_rel, ewv_ref[...], preferred_element_type=jnp.float32)
        + cnt * (tbv_ref[...] + ebv_ref[...]))


def kernel(nid, k_, q_, v_, start_t, end_t, neighbors, times, rels,
           t2v_w0, t2v_b0, t2v_W, t2v_B,
           tWk, tbk, tWq, tbq, tWv, tbv,
           eWk, ebk, eWq, ebq, eWv, ebv):
    del nid, k_, q_, tWk, tbk, tWq, tbq, eWk, ebk, eWq, ebq
    s_f = jnp.asarray(start_t, jnp.float32)
    e_f = jnp.asarray(end_t, jnp.float32)
    t_flat = jnp.reshape(times, (N_NBR,)).astype(jnp.float32)
    nbr = jnp.reshape(neighbors, (N_NBR,)).astype(jnp.int32)
    se16 = jnp.broadcast_to(
        jnp.stack([s_f, e_f])[:, None], (2, LANES))

    scpart = _sc_gather_sum(v_, nbr, t_flat, se16)

    wfull = jnp.concatenate(
        [jnp.zeros((1, 1), jnp.float32), t2v_W], axis=1)
    bfull = jnp.concatenate(
        [jnp.zeros((1,), jnp.float32), t2v_B], axis=0)[None, :]
    wb = jnp.concatenate([wfull, bfull], axis=0)
    params = jnp.stack(
        [s_f, e_f, t2v_w0[0, 0], t2v_b0[0]]).reshape(1, 4)

    tc_out = pl.pallas_call(
        _tc_body,
        out_shape=jax.ShapeDtypeStruct((1, HIDDEN), jnp.float32),
    )(jnp.reshape(times, (1, N_NBR)), jnp.reshape(times, (64, 128)),
      jnp.transpose(rels), wb, tWv, eWv, tbv[None, :], ebv[None, :], params)
    return jnp.sum(jnp.concatenate([scpart, tc_out], axis=0),
                   axis=0, keepdims=True)
